# initial kernel scaffold (unmeasured)
import jax
import jax.numpy as jnp
from jax import lax
from jax.experimental import pallas as pl
from jax.experimental.pallas import tpu as pltpu


def kernel(
    x,
):
    def body(*refs):
        pass

    out_shape = jax.ShapeDtypeStruct(..., jnp.float32)
    return pl.pallas_call(body, out_shape=out_shape)(...)


# baseline (device time: 29181 ns/iter reference)
import jax
import jax.numpy as jnp
from jax import lax
from jax.experimental import pallas as pl
from jax.experimental.pallas import tpu as pltpu

K = 16


def _topk_desc(v, k):
    cols = []
    for i in range(k):
        mi = jnp.max(v, axis=1, keepdims=True)
        cols.append(mi)
        if i < k - 1:
            v = jnp.where(v == mi, -jnp.inf, v)
    return jnp.concatenate(cols, axis=1)


def kernel(x):
    m, n_loc = x.shape

    def body(x_ref, o_ref, cand_ref, peer_ref, send_sem, recv_sem):
        my_x = lax.axis_index("x")
        my_y = lax.axis_index("y")

        cand_ref[:, :] = _topk_desc(x_ref[:, :], K)

        rdma = pltpu.make_async_remote_copy(
            src_ref=cand_ref,
            dst_ref=peer_ref,
            send_sem=send_sem,
            recv_sem=recv_sem,
            device_id=(1 - my_x, my_y),
            device_id_type=pl.DeviceIdType.MESH,
        )
        rdma.start()
        rdma.wait()

        allc = jnp.concatenate([cand_ref[:, :], peer_ref[:, :]], axis=1)
        o_ref[:, :] = _topk_desc(allc, K)

    return pl.pallas_call(
        body,
        out_shape=jax.ShapeDtypeStruct((m, K), jnp.float32),
        in_specs=[pl.BlockSpec(memory_space=pltpu.VMEM)],
        out_specs=pl.BlockSpec(memory_space=pltpu.VMEM),
        scratch_shapes=[
            pltpu.VMEM((m, K), jnp.float32),
            pltpu.VMEM((m, K), jnp.float32),
            pltpu.SemaphoreType.DMA,
            pltpu.SemaphoreType.DMA,
        ],
    )(x)


# device time: 25924 ns/iter; 1.1256x vs baseline; 1.1256x over previous
import jax
import jax.numpy as jnp
from jax import lax
from jax.experimental import pallas as pl
from jax.experimental.pallas import tpu as pltpu

K = 16


def _topk_desc(v, k):
    cols = []
    for i in range(k):
        mi = jnp.max(v, axis=1, keepdims=True)
        cols.append(mi)
        if i < k - 1:
            v = jnp.where(v == mi, -jnp.inf, v)
    return jnp.concatenate(cols, axis=1)


def kernel(x):
    m, n_loc = x.shape
    n_half = n_loc // 2

    def body(x_ref, o_ref, a_ref, b_ref, c_ref, d_ref, sems):
        my_x = lax.axis_index("x")
        my_y = lax.axis_index("y")

        barrier = pltpu.get_barrier_semaphore()
        pl.semaphore_signal(
            barrier, inc=1, device_id=(my_x, 1 - my_y),
            device_id_type=pl.DeviceIdType.MESH,
        )
        pl.semaphore_signal(
            barrier, inc=1, device_id=(1 - my_x, my_y),
            device_id_type=pl.DeviceIdType.MESH,
        )
        pl.semaphore_wait(barrier, 2)

        a_ref[:, :] = _topk_desc(x_ref[:, pl.ds(my_y * n_half, n_half)], K)

        ph1 = pltpu.make_async_remote_copy(
            src_ref=a_ref,
            dst_ref=b_ref,
            send_sem=sems.at[0],
            recv_sem=sems.at[1],
            device_id=(my_x, 1 - my_y),
            device_id_type=pl.DeviceIdType.MESH,
        )
        ph1.start()
        ph1.wait()
        c_ref[:, :] = _topk_desc(
            jnp.concatenate([a_ref[:, :], b_ref[:, :]], axis=1), K
        )

        ph2 = pltpu.make_async_remote_copy(
            src_ref=c_ref,
            dst_ref=d_ref,
            send_sem=sems.at[2],
            recv_sem=sems.at[3],
            device_id=(1 - my_x, my_y),
            device_id_type=pl.DeviceIdType.MESH,
        )
        ph2.start()
        ph2.wait()
        o_ref[:, :] = _topk_desc(
            jnp.concatenate([c_ref[:, :], d_ref[:, :]], axis=1), K
        )

    return pl.pallas_call(
        body,
        out_shape=jax.ShapeDtypeStruct((m, K), jnp.float32),
        in_specs=[pl.BlockSpec(memory_space=pltpu.VMEM)],
        out_specs=pl.BlockSpec(memory_space=pltpu.VMEM),
        scratch_shapes=[
            pltpu.VMEM((m, K), jnp.float32),
            pltpu.VMEM((m, K), jnp.float32),
            pltpu.VMEM((m, K), jnp.float32),
            pltpu.VMEM((m, K), jnp.float32),
            pltpu.SemaphoreType.DMA((4,)),
        ],
        compiler_params=pltpu.CompilerParams(collective_id=0),
    )(x)


# device time: 23353 ns/iter; 1.2496x vs baseline; 1.1101x over previous
import jax
import jax.numpy as jnp
from jax import lax
from jax.experimental import pallas as pl
from jax.experimental.pallas import tpu as pltpu

K = 16


def _topk_desc(v, k):
    cols = []
    for i in range(k):
        mi = jnp.max(v, axis=1, keepdims=True)
        cols.append(mi)
        if i < k - 1:
            v = jnp.where(v == mi, -jnp.inf, v)
    return jnp.concatenate(cols, axis=1)


def kernel(x):
    m, n_loc = x.shape
    n_half = n_loc // 2

    def body(x_ref, o_ref, a_ref, recv_ref, send_sems, recv_sems):
        my_x = lax.axis_index("x")
        my_y = lax.axis_index("y")
        peers = [
            (my_x, 1 - my_y),
            (1 - my_x, my_y),
            (1 - my_x, 1 - my_y),
        ]

        barrier = pltpu.get_barrier_semaphore()
        for p in peers:
            pl.semaphore_signal(
                barrier, inc=1, device_id=p,
                device_id_type=pl.DeviceIdType.MESH,
            )
        pl.semaphore_wait(barrier, 3)

        a_ref[:, :] = _topk_desc(x_ref[:, pl.ds(my_y * n_half, n_half)], K)

        rdmas = []
        for i, p in enumerate(peers):
            r = pltpu.make_async_remote_copy(
                src_ref=a_ref,
                dst_ref=recv_ref.at[i],
                send_sem=send_sems.at[i],
                recv_sem=recv_sems.at[i],
                device_id=p,
                device_id_type=pl.DeviceIdType.MESH,
            )
            r.start()
            rdmas.append(r)
        for r in rdmas:
            r.wait()

        allc = jnp.concatenate(
            [a_ref[:, :], recv_ref[0], recv_ref[1], recv_ref[2]], axis=1
        )
        o_ref[:, :] = _topk_desc(allc, K)

    return pl.pallas_call(
        body,
        out_shape=jax.ShapeDtypeStruct((m, K), jnp.float32),
        in_specs=[pl.BlockSpec(memory_space=pltpu.VMEM)],
        out_specs=pl.BlockSpec(memory_space=pltpu.VMEM),
        scratch_shapes=[
            pltpu.VMEM((m, K), jnp.float32),
            pltpu.VMEM((3, m, K), jnp.float32),
            pltpu.SemaphoreType.DMA((3,)),
            pltpu.SemaphoreType.DMA((3,)),
        ],
        compiler_params=pltpu.CompilerParams(collective_id=0),
    )(x)
